# baseline (speedup 1.0000x reference)
import jax
import jax.numpy as jnp
from jax.experimental import pallas as pl
from jax.experimental.pallas import tpu as pltpu

T, C, H, E, I = 2048, 1024, 64, 8, 256
BLK = 256
GRID = T // BLK

PREC = jax.lax.Precision.DEFAULT


def _dot(a, b, precision=PREC):
    return jax.lax.dot_general(a, b, (((1,), (0,)), ((), ())),
                               precision=precision,
                               preferred_element_type=jnp.float32)


def _gating(hn, sn, gb):
    logits = _dot(hn, sn) - gb
    gated = jnp.maximum(logits, 0.0)
    iot = jax.lax.broadcasted_iota(jnp.int32, (BLK, E), 1)
    m1 = jnp.max(logits, axis=1, keepdims=True)
    i1 = jnp.min(jnp.where(logits == m1, iot, E), axis=1, keepdims=True)
    sel1 = iot == i1
    l2 = jnp.where(sel1, -1e30, logits)
    m2 = jnp.max(l2, axis=1, keepdims=True)
    i2 = jnp.min(jnp.where(l2 == m2, iot, E), axis=1, keepdims=True)
    fb = (sel1 | (iot == i2)).astype(jnp.float32)
    thr = (logits > 0.0).astype(jnp.float32)
    inactive = m1 <= 0.0
    mask = jnp.where(inactive, fb, thr)
    gm = jnp.where(mask > 0.0, gated, -1e9)
    ex = jnp.exp(gm - jnp.max(gm, axis=1, keepdims=True))
    probs = ex / jnp.sum(ex, axis=1, keepdims=True)
    return probs * mask


def _ln(x, w, b):
    mu = jnp.mean(x, axis=1, keepdims=True)
    xc = x - mu
    var = jnp.mean(xc * xc, axis=1, keepdims=True)
    return xc / jnp.sqrt(var + 1e-5) * w + b


def _mix(z, w, width):
    acc = z[:, 0:width] * w[:, 0:1]
    for e in range(1, E):
        acc = acc + z[:, e * width:(e + 1) * width] * w[:, e:e + 1]
    return acc


def _body(h_ref, rn_ref, hs_ref, sna_ref, gba_ref, wq_ref, wk_ref, wv_ref,
          cos_ref, sin_ref, omat_ref, ln2w_ref, ln2b_ref, snm_ref, gbm_ref,
          upmat_ref, downmat_ref, out_ref, ket_s, v_s):
    i = pl.program_id(0)
    h = h_ref[...]
    w = _gating(h / rn_ref[...], sna_ref[...], gba_ref[...])
    q = _mix(_dot(h, wq_ref[...]), w, H)
    k = _mix(_dot(h, wk_ref[...]), w, H)
    v = _mix(_dot(h, wv_ref[...]), w, H)
    cos = cos_ref[...]
    sin = sin_ref[...]

    def rope(t):
        rot = jnp.concatenate([-t[:, H // 2:], t[:, :H // 2]], axis=1)
        return t * cos + rot * sin

    qe = rope(q)

    @pl.when(i == 0)
    def _():
        v_s[...] = jnp.zeros((T, H), jnp.float32)

    ket_s[:, pl.ds(i * BLK, BLK)] = rope(k).T
    v_s[pl.ds(i * BLK, BLK), :] = v

    s = _dot(qe, ket_s[...]) * 0.125
    row = i * BLK + jax.lax.broadcasted_iota(jnp.int32, (BLK, T), 0)
    col = jax.lax.broadcasted_iota(jnp.int32, (BLK, T), 1)
    s = jnp.where(col <= row, s, -1e9)
    KT = T // 2
    vfull = v_s[...]
    s1, s2 = s[:, :KT], s[:, KT:]
    m1 = jnp.max(s1, axis=1, keepdims=True)
    p1 = jnp.exp(s1 - m1)
    den1 = jnp.sum(p1, axis=1, keepdims=True)
    ao1 = _dot(p1, vfull[:KT]) * (1.0 / den1)
    m2 = jnp.maximum(m1, jnp.max(s2, axis=1, keepdims=True))
    corr = jnp.where(m1 == m2, 0.0, m1 - m2)
    scale = jnp.exp(corr) * den1
    p2 = jnp.exp(s2 - m2)
    den2 = scale + jnp.sum(p2, axis=1, keepdims=True)
    ao = (_dot(p2, vfull[KT:]) + scale * ao1) * (1.0 / den2)

    wao = jnp.concatenate([ao * w[:, e:e + 1] for e in range(E)], axis=1)
    h1 = hs_ref[...] + _dot(wao, omat_ref[...])
    h2 = _ln(h1, ln2w_ref[...], ln2b_ref[...])
    hn2 = h2 / jnp.maximum(jnp.sqrt(jnp.sum(h2 * h2, axis=1, keepdims=True)),
                           1e-12)
    w2 = _gating(hn2, snm_ref[...], gbm_ref[...])
    moe = None
    for e in range(E):
        up = _dot(h2, upmat_ref[e])
        up = up * jax.nn.sigmoid(up)
        part = _dot(up * w2[:, e:e + 1], downmat_ref[e])
        moe = part if moe is None else moe + part
    out_ref[...] = h1 + moe


def kernel(hidden_states, ln1_w, ln1_b, sim_a, gates_a, q_proj, k_proj,
           v_proj, o_proj, ln2_w, ln2_b, sim_m, gates_m, up_proj, down_proj,
           position_ids):
    hs = hidden_states.reshape(T, C)
    h = _ln(hs, ln1_w.reshape(1, C), ln1_b.reshape(1, C))
    rn = jnp.maximum(jnp.sqrt(jnp.sum(h * h, axis=1, keepdims=True)), 1e-12)
    sna = sim_a / jnp.maximum(jnp.sqrt(jnp.sum(sim_a * sim_a, axis=0,
                                               keepdims=True)), 1e-12)
    snm = sim_m / jnp.maximum(jnp.sqrt(jnp.sum(sim_m * sim_m, axis=0,
                                               keepdims=True)), 1e-12)
    gba = jax.nn.sigmoid(gates_a).reshape(1, E)
    gbm = jax.nn.sigmoid(gates_m).reshape(1, E)
    wq = q_proj.transpose(1, 0, 2).reshape(C, E * H)
    wk = k_proj.transpose(1, 0, 2).reshape(C, E * H)
    wv = v_proj.transpose(1, 0, 2).reshape(C, E * H)
    omat = o_proj.reshape(E * H, C)

    pos = position_ids.reshape(T).astype(jnp.float32)
    inv_freq = 1.0 / (10000.0 ** (jnp.arange(0, H, 2, dtype=jnp.float32) / H))
    fr = pos[:, None] * inv_freq[None, :]
    emb = jnp.concatenate([fr, fr], axis=-1)
    cos, sin = jnp.cos(emb), jnp.sin(emb)

    f32 = jnp.float32
    const = lambda shape: pl.BlockSpec(shape, lambda i: (0, 0))
    const3 = lambda shape: pl.BlockSpec(shape, lambda i: (0, 0, 0))
    blk = lambda shape: pl.BlockSpec(shape, lambda i: (i, 0))

    out = pl.pallas_call(
        _body,
        grid=(GRID,),
        in_specs=[blk((BLK, C)), blk((BLK, 1)), blk((BLK, C)), const((C, E)),
                  const((1, E)), const((C, E * H)), const((C, E * H)),
                  const((C, E * H)), blk((BLK, H)), blk((BLK, H)),
                  const((E * H, C)), const((1, C)), const((1, C)),
                  const((C, E)), const((1, E)), const3((E, C, I)),
                  const3((E, I, C))],
        out_specs=blk((BLK, C)),
        out_shape=jax.ShapeDtypeStruct((T, C), f32),
        scratch_shapes=[pltpu.VMEM((H, T), f32), pltpu.VMEM((T, H), f32)],
        compiler_params=pltpu.CompilerParams(
            dimension_semantics=("arbitrary",),
            vmem_limit_bytes=100 * 1024 * 1024),
    )(h, rn, hs, sna, gba, wq, wk, wv, cos, sin, omat,
      ln2_w.reshape(1, C), ln2_b.reshape(1, C), snm, gbm, up_proj, down_proj)

    return out.reshape(1, T, C)
